# baseline (speedup 1.0000x reference)
import jax
import jax.numpy as jnp
from jax import lax
from jax.experimental import pallas as pl
from jax.experimental.pallas import tpu as pltpu

_NUM_CLASSES = 80
_NUM_ANCHORS = 3
_STRIDE = 8.0
_NCH = _NUM_CLASSES + 7


def _yolo_body(x_ref, scale_ref, o_ref, *, rb, g):
    r = pl.program_id(2)
    v = x_ref[0, 0]
    t = v.T
    sig = jax.nn.sigmoid(t)
    ex = jnp.exp(t)
    c = lax.broadcasted_iota(jnp.int32, t.shape, 1)
    n = r * rb + lax.broadcasted_iota(jnp.int32, t.shape, 0)
    gx = (n % g).astype(jnp.float32)
    gy = (n // g).astype(jnp.float32)
    grid_off = jnp.where(c == 0, gx, gy)
    xy = (sig + grid_off) * _STRIDE
    wh = ex * scale_ref[0, 0]
    out = jnp.where(c < 2, xy,
          jnp.where(c < 4, wh,
          jnp.where(c < 6, t, sig)))
    o_ref[0, 0] = out


def kernel(x, anchors):
    B, C, G, _ = x.shape
    nA, nCh = _NUM_ANCHORS, _NCH
    GG = G * G
    rb = 1024
    xr = x.reshape(B, nA, nCh, GG)
    scale = jnp.ones((nA, nCh), dtype=jnp.float32)
    scale = scale.at[:, 2].set(anchors[:, 0]).at[:, 3].set(anchors[:, 1])
    scale = scale.reshape(nA, 1, nCh)

    import functools
    body = functools.partial(_yolo_body, rb=rb, g=G)
    out = pl.pallas_call(
        body,
        grid=(B, nA, GG // rb),
        in_specs=[
            pl.BlockSpec((1, 1, nCh, rb), lambda b, a, r: (b, a, 0, r)),
            pl.BlockSpec((1, 1, nCh), lambda b, a, r: (a, 0, 0)),
        ],
        out_specs=pl.BlockSpec((1, 1, rb, nCh), lambda b, a, r: (b, a, r, 0)),
        out_shape=jax.ShapeDtypeStruct((B, nA, GG, nCh), jnp.float32),
        compiler_params=pltpu.CompilerParams(
            dimension_semantics=("parallel", "parallel", "parallel"),
        ),
    )(xr, scale)
    return out.reshape(B, nA * GG, nCh)


# speedup vs baseline: 1.2166x; 1.2166x over previous
import jax
import jax.numpy as jnp
from jax import lax
from jax.experimental import pallas as pl
from jax.experimental.pallas import tpu as pltpu

_NUM_CLASSES = 80
_NUM_ANCHORS = 3
_STRIDE = 8.0
_NCH = _NUM_CLASSES + 7


def _yolo_body(x_ref, scale_ref, o_ref, *, rb, g):
    r = pl.program_id(2)
    v = x_ref[0, 0]
    t = v.T
    sig = jax.nn.sigmoid(t)
    ex = jnp.exp(t)
    c = lax.broadcasted_iota(jnp.int32, t.shape, 1)
    n = r * rb + lax.broadcasted_iota(jnp.int32, t.shape, 0)
    gx = (n % g).astype(jnp.float32)
    gy = (n // g).astype(jnp.float32)
    grid_off = jnp.where(c == 0, gx, gy)
    xy = (sig + grid_off) * _STRIDE
    wh = ex * scale_ref[0, 0]
    out = jnp.where(c < 2, xy,
          jnp.where(c < 4, wh,
          jnp.where(c < 6, t, sig)))
    o_ref[0, 0] = out


def kernel(x, anchors):
    B, C, G, _ = x.shape
    nA, nCh = _NUM_ANCHORS, _NCH
    GG = G * G
    rb = 4096
    xr = x.reshape(B, nA, nCh, GG)
    scale = jnp.ones((nA, nCh), dtype=jnp.float32)
    scale = scale.at[:, 2].set(anchors[:, 0]).at[:, 3].set(anchors[:, 1])
    scale = scale.reshape(nA, 1, nCh)

    import functools
    body = functools.partial(_yolo_body, rb=rb, g=G)
    out = pl.pallas_call(
        body,
        grid=(B, nA, GG // rb),
        in_specs=[
            pl.BlockSpec((1, 1, nCh, rb), lambda b, a, r: (b, a, 0, r)),
            pl.BlockSpec((1, 1, nCh), lambda b, a, r: (a, 0, 0)),
        ],
        out_specs=pl.BlockSpec((1, 1, rb, nCh), lambda b, a, r: (b, a, r, 0)),
        out_shape=jax.ShapeDtypeStruct((B, nA, GG, nCh), jnp.float32),
        compiler_params=pltpu.CompilerParams(
            dimension_semantics=("parallel", "parallel", "parallel"),
        ),
    )(xr, scale)
    return out.reshape(B, nA * GG, nCh)


# speedup vs baseline: 1.2391x; 1.0185x over previous
import jax
import jax.numpy as jnp
from jax.experimental import pallas as pl
from jax.experimental.pallas import tpu as pltpu

_NUM_CLASSES = 80
_NUM_ANCHORS = 3
_STRIDE = 8.0
_NCH = _NUM_CLASSES + 7


def _yolo_body(x_ref, rowc_ref, gxy_ref, o_ref):
    v = x_ref[0, 0]
    rc = rowc_ref[0]
    sgn = rc[:, 0:1]
    ca = rc[:, 1:2]
    cb = rc[:, 2:3]
    cc = rc[:, 3:4]
    e = jnp.exp(v * sgn)
    sig = 1.0 / (1.0 + e)
    w = ca * sig + cb * e + cc * v + gxy_ref[...]
    o_ref[0, 0] = w.T


def kernel(x, anchors):
    B, C, G, _ = x.shape
    nA, nCh = _NUM_ANCHORS, _NCH
    GG = G * G
    xr = x.reshape(B, nA, nCh, GG)

    rows = jnp.arange(nCh)
    sgn = jnp.where((rows == 2) | (rows == 3), 1.0, -1.0)
    ca = jnp.where(rows < 2, _STRIDE, jnp.where(rows >= 6, 1.0, 0.0))
    cc = jnp.where((rows == 4) | (rows == 5), 1.0, 0.0)
    cb = jnp.zeros((nA, nCh), jnp.float32)
    cb = cb.at[:, 2].set(anchors[:, 0]).at[:, 3].set(anchors[:, 1])
    rowc = jnp.stack(
        [jnp.broadcast_to(sgn, (nA, nCh)),
         jnp.broadcast_to(ca, (nA, nCh)),
         cb,
         jnp.broadcast_to(cc, (nA, nCh))], axis=-1
    ).astype(jnp.float32)

    n = jnp.arange(GG, dtype=jnp.int32)
    gxy2 = jnp.stack([(n % G), (n // G)]).astype(jnp.float32) * _STRIDE
    gxy = jnp.zeros((nCh, GG), jnp.float32).at[0:2].set(gxy2)

    out = pl.pallas_call(
        _yolo_body,
        grid=(B, nA),
        in_specs=[
            pl.BlockSpec((1, 1, nCh, GG), lambda b, a: (b, a, 0, 0)),
            pl.BlockSpec((1, nCh, 4), lambda b, a: (a, 0, 0)),
            pl.BlockSpec((nCh, GG), lambda b, a: (0, 0)),
        ],
        out_specs=pl.BlockSpec((1, 1, GG, nCh), lambda b, a: (b, a, 0, 0)),
        out_shape=jax.ShapeDtypeStruct((B, nA, GG, nCh), jnp.float32),
        compiler_params=pltpu.CompilerParams(
            dimension_semantics=("parallel", "parallel"),
        ),
    )(xr, rowc, gxy)
    return out.reshape(B, nA * GG, nCh)


# speedup vs baseline: 1.5215x; 1.2278x over previous
import jax
import jax.numpy as jnp
from jax.experimental import pallas as pl
from jax.experimental.pallas import tpu as pltpu

_NUM_CLASSES = 80
_NUM_ANCHORS = 3
_STRIDE = 8.0
_NCH = _NUM_CLASSES + 7


def _yolo_body(anch_ref, x_ref, o_ref):
    a = pl.program_id(1)
    aw = anch_ref[a, 0]
    ah = anch_ref[a, 1]
    v = x_ref[0]
    c = jax.lax.broadcasted_iota(jnp.int32, v.shape, 0)
    gx = jax.lax.broadcasted_iota(jnp.int32, v.shape, 2).astype(jnp.float32)
    gy = jax.lax.broadcasted_iota(jnp.int32, v.shape, 1).astype(jnp.float32)
    is_wh = (c == 2) | (c == 3)
    sgn = jnp.where(is_wh, 1.0, -1.0)
    ca = jnp.where(c < 2, _STRIDE, jnp.where(c >= 6, 1.0, 0.0))
    cb = jnp.where(c == 2, aw, jnp.where(c == 3, ah, 0.0))
    cc = jnp.where((c == 4) | (c == 5), 1.0, 0.0)
    add = _STRIDE * jnp.where(c == 0, gx, jnp.where(c == 1, gy, 0.0))
    e = jnp.exp(v * sgn)
    sig = 1.0 / (1.0 + e)
    w = ca * sig + cb * e + cc * v + add
    o_ref[0, 0] = jnp.transpose(w, (1, 2, 0))


def kernel(x, anchors):
    B, C, G, _ = x.shape
    nA, nCh = _NUM_ANCHORS, _NCH

    out = pl.pallas_call(
        _yolo_body,
        grid=(B, nA),
        in_specs=[
            pl.BlockSpec(memory_space=pltpu.SMEM),
            pl.BlockSpec((1, nCh, G, G), lambda b, a: (b, a, 0, 0)),
        ],
        out_specs=pl.BlockSpec((1, 1, G, G, nCh), lambda b, a: (b, a, 0, 0, 0)),
        out_shape=jax.ShapeDtypeStruct((B, nA, G, G, nCh), jnp.float32),
        compiler_params=pltpu.CompilerParams(
            dimension_semantics=("parallel", "parallel"),
        ),
    )(anchors, x)
    return out.reshape(B, nA * G * G, nCh)


# speedup vs baseline: 1.5750x; 1.0352x over previous
import jax
import jax.numpy as jnp
from jax.experimental import pallas as pl
from jax.experimental.pallas import tpu as pltpu

_NUM_CLASSES = 80
_NUM_ANCHORS = 3
_STRIDE = 8.0
_NCH = _NUM_CLASSES + 7


def _yolo_body(anch_ref, x_ref, o_ref):
    for i in range(_NUM_ANCHORS):
        aw = anch_ref[i, 0]
        ah = anch_ref[i, 1]
        v = x_ref[0, i * _NCH:(i + 1) * _NCH]
        c = jax.lax.broadcasted_iota(jnp.int32, v.shape, 0)
        gx = jax.lax.broadcasted_iota(jnp.int32, v.shape, 2).astype(jnp.float32)
        gy = jax.lax.broadcasted_iota(jnp.int32, v.shape, 1).astype(jnp.float32)
        sgn = jnp.where((c == 2) | (c == 3), 1.0, -1.0)
        ca = jnp.where(c < 2, _STRIDE, jnp.where(c >= 6, 1.0, 0.0))
        cb = jnp.where(c == 2, aw, jnp.where(c == 3, ah, 0.0))
        cc = jnp.where((c == 4) | (c == 5), 1.0, 0.0)
        add = _STRIDE * jnp.where(c == 0, gx, jnp.where(c == 1, gy, 0.0))
        e = jnp.exp(v * sgn)
        sig = 1.0 / (1.0 + e)
        w = ca * sig + cb * e + cc * v + add
        o_ref[0, i] = jnp.transpose(w, (1, 2, 0))


def kernel(x, anchors):
    B, C, G, _ = x.shape
    nA, nCh = _NUM_ANCHORS, _NCH

    out = pl.pallas_call(
        _yolo_body,
        grid=(B,),
        in_specs=[
            pl.BlockSpec(memory_space=pltpu.SMEM),
            pl.BlockSpec((1, C, G, G), lambda b: (b, 0, 0, 0)),
        ],
        out_specs=pl.BlockSpec((1, nA, G, G, nCh), lambda b: (b, 0, 0, 0, 0)),
        out_shape=jax.ShapeDtypeStruct((B, nA, G, G, nCh), jnp.float32),
        compiler_params=pltpu.CompilerParams(
            dimension_semantics=("arbitrary",),
        ),
    )(anchors, x)
    return out.reshape(B, nA * G * G, nCh)
